# baseline (speedup 1.0000x reference)
import functools

import jax
import jax.numpy as jnp
from jax import lax
from jax.experimental import pallas as pl
from jax.experimental.pallas import tpu as pltpu
from jax.experimental.pallas import tpu_sc as plsc

NUM_FEATURE = 26
EMBED_K = 16
BATCH = 16384

NC = 2
NS = 16
L = 16
NW = NC * NS
BPW = BATCH // NW
NGRP = BPW // L


def _sc_body(x_hbm, f_hbm, w_hbm, h_hbm, p_hbm, out1_hbm, out2_hbm,
             xbuf, uidx, vidx, urows, vrows, fbuf, pbuf,
             o1buf, o2buf, sem_f, sem_u, sem_v):
    wid = lax.axis_index("s") * NC + lax.axis_index("c")
    base = wid * BPW

    f_cp = pltpu.async_copy(f_hbm.at[pl.ds(base, BPW)], fbuf, sem_f)
    pltpu.sync_copy(x_hbm.at[pl.ds(base, BPW)], xbuf)
    pltpu.sync_copy(p_hbm, pbuf)

    lanes = lax.iota(jnp.int32, L)
    zeros = jnp.zeros((L,), jnp.int32)
    ones = jnp.ones((L,), jnp.int32)

    def deint(g, _):
        jvec = g * L + lanes
        uidx[pl.ds(g * L, L)] = plsc.load_gather(xbuf, [jvec, zeros])
        vidx[pl.ds(g * L, L)] = plsc.load_gather(xbuf, [jvec, ones])
        return _

    lax.fori_loop(0, NGRP, deint, None)

    u_cp = pltpu.async_copy(w_hbm.at[uidx], urows, sem_u)
    v_cp = pltpu.async_copy(h_hbm.at[vidx], vrows, sem_v)
    f_cp.wait()
    u_cp.wait()
    v_cp.wait()

    dwa = pbuf[pl.ds(0, L)]
    dwb = pbuf[pl.ds(L, L)]
    dbv = pbuf[pl.ds(2 * L, L)]

    def compute(g, _):
        jvec = g * L + lanes
        acc1 = jnp.zeros((L,), jnp.float32)
        for k in range(EMBED_K):
            ksplat = jnp.full((L,), k, jnp.int32)
            gu = plsc.load_gather(urows, [jvec, ksplat])
            gv = plsc.load_gather(vrows, [jvec, ksplat])
            acc1 = acc1 + gu * gv
        acc2 = dbv
        for k in range(NUM_FEATURE):
            ksplat = jnp.full((L,), k, jnp.int32)
            gf = plsc.load_gather(fbuf, [jvec, ksplat])
            wk = dwa[k] if k < L else dwb[k - L]
            acc2 = acc2 + gf * wk
        o2 = jnp.minimum(jnp.maximum(jnp.exp(acc2), 1e-05), 3.0)
        o1buf[pl.ds(g * L, L)] = acc1
        o2buf[pl.ds(g * L, L)] = o2
        return _

    lax.fori_loop(0, NGRP, compute, None)

    pltpu.sync_copy(o1buf, out1_hbm.at[pl.ds(base, BPW)])
    pltpu.sync_copy(o2buf, out2_hbm.at[pl.ds(base, BPW)])


@jax.jit
def kernel(x, feature, W, H, Dw, Db):
    params = jnp.concatenate([
        Dw[:, 0],
        jnp.zeros((2 * L - NUM_FEATURE,), jnp.float32),
        jnp.broadcast_to(Db, (L,)),
    ])
    mesh = plsc.VectorSubcoreMesh(core_axis_name="c", subcore_axis_name="s",
                                  num_cores=NC, num_subcores=NS)
    out1, out2 = pl.kernel(
        _sc_body,
        out_type=[
            jax.ShapeDtypeStruct((BATCH,), jnp.float32),
            jax.ShapeDtypeStruct((BATCH,), jnp.float32),
        ],
        mesh=mesh,
        compiler_params=pltpu.CompilerParams(needs_layout_passes=False,
                                             use_tc_tiling_on_sc=False),
        scratch_types=[
            pltpu.VMEM((BPW, 2), jnp.int32),
            pltpu.VMEM((BPW,), jnp.int32),
            pltpu.VMEM((BPW,), jnp.int32),
            pltpu.VMEM((BPW, EMBED_K), jnp.float32),
            pltpu.VMEM((BPW, EMBED_K), jnp.float32),
            pltpu.VMEM((BPW, NUM_FEATURE), jnp.float32),
            pltpu.VMEM((3 * L,), jnp.float32),
            pltpu.VMEM((BPW,), jnp.float32),
            pltpu.VMEM((BPW,), jnp.float32),
            pltpu.SemaphoreType.DMA,
            pltpu.SemaphoreType.DMA,
            pltpu.SemaphoreType.DMA,
        ],
    )(x, feature, W, H, params)
    return out1, out2
